# Initial kernel scaffold; baseline (speedup 1.0000x reference)
import jax
import jax.numpy as jnp
from jax.experimental import pallas as pl


def kernel(xyz, points, W0, b0, g0, be0, W1, b1, g1, be1, W2, b2, g2, be2):
    raise NotImplementedError("write your pallas kernel here")


# speedup vs baseline: 2.0212x; 2.0212x over previous
import functools

import jax
import jax.numpy as jnp
import numpy as np
from jax.experimental import pallas as pl
from jax.experimental.pallas import tpu as pltpu

B = 16
N = 2048
D = 64
S = 512
K = 32
CIN = 67
NTOT = float(B * K * S)
EPS = 1e-5


def _fps_body(xyz_ref, far0_ref, cent_ref, nx_ref, ny_ref, nz_ref, qc_ref):
    X = xyz_ref[:, 0, :]
    Y = xyz_ref[:, 1, :]
    Z = xyz_ref[:, 2, :]
    jlane = jax.lax.broadcasted_iota(jnp.int32, (B, N), 1)
    siota = jax.lax.broadcasted_iota(jnp.int32, (B, S), 1)
    far0 = far0_ref[...]

    dist0 = jnp.full((B, N), 1e10, dtype=jnp.float32)
    cent0 = jnp.zeros((B, S), dtype=jnp.int32)
    zf = jnp.zeros((B, S), dtype=jnp.float32)

    def body(i, c):
        dist, far, cent, nx, ny, nz, qc = c
        mask = jlane == far
        cx = jnp.sum(jnp.where(mask, X, 0.0), axis=1, keepdims=True)
        cy = jnp.sum(jnp.where(mask, Y, 0.0), axis=1, keepdims=True)
        cz = jnp.sum(jnp.where(mask, Z, 0.0), axis=1, keepdims=True)
        sel = siota == i
        cent = jnp.where(sel, far, cent)
        nx = jnp.where(sel, cx, nx)
        ny = jnp.where(sel, cy, ny)
        nz = jnp.where(sel, cz, nz)
        qc = jnp.where(sel, (cx * cx + cy * cy) + cz * cz, qc)
        dx = X - cx
        dy = Y - cy
        dz = Z - cz
        d = dx * dx + dy * dy + dz * dz
        dist = jnp.where(d < dist, d, dist)
        m = jnp.max(dist, axis=1, keepdims=True)
        far = jnp.min(jnp.where(dist == m, jlane, N), axis=1, keepdims=True)
        return (dist, far, cent, nx, ny, nz, qc)

    _, _, cent, nx, ny, nz, qc = jax.lax.fori_loop(
        0, S, body, (dist0, far0, cent0, zf, zf, zf, zf))
    cent_ref[...] = cent
    nx_ref[...] = nx
    ny_ref[...] = ny
    nz_ref[...] = nz
    qc_ref[...] = qc


def _run_fps(xyz, far0):
    outs = (
        jax.ShapeDtypeStruct((B, S), jnp.int32),
        jax.ShapeDtypeStruct((B, S), jnp.float32),
        jax.ShapeDtypeStruct((B, S), jnp.float32),
        jax.ShapeDtypeStruct((B, S), jnp.float32),
        jax.ShapeDtypeStruct((B, S), jnp.float32),
    )
    return pl.pallas_call(_fps_body, out_shape=outs)(xyz, far0)


def _dist_body(xyz_ref, nx_ref, ny_ref, nz_ref, qc_ref, d_ref):
    X = xyz_ref[0, 0, :][None, :]
    Y = xyz_ref[0, 1, :][None, :]
    Z = xyz_ref[0, 2, :][None, :]
    qj = (X * X + Y * Y) + Z * Z
    cx = nx_ref[0, 0, :][:, None]
    cy = ny_ref[0, 0, :][:, None]
    cz = nz_ref[0, 0, :][:, None]
    qc = qc_ref[0, 0, :][:, None]
    bf = jnp.bfloat16
    f32 = jnp.float32
    Xb = X.astype(bf).astype(f32)
    Yb = Y.astype(bf).astype(f32)
    Zb = Z.astype(bf).astype(f32)
    cxb = cx.astype(bf).astype(f32)
    cyb = cy.astype(bf).astype(f32)
    czb = cz.astype(bf).astype(f32)
    inner = (cxb * Xb + cyb * Yb) + czb * Zb
    d_ref[0] = (-2.0 * inner + qj) + qc


def _run_dist(xyz, nx, ny, nz, qc):
    return pl.pallas_call(
        _dist_body,
        grid=(B,),
        in_specs=[
            pl.BlockSpec((1, 3, N), lambda b: (b, 0, 0)),
            pl.BlockSpec((1, 1, S), lambda b: (b, 0, 0)),
            pl.BlockSpec((1, 1, S), lambda b: (b, 0, 0)),
            pl.BlockSpec((1, 1, S), lambda b: (b, 0, 0)),
            pl.BlockSpec((1, 1, S), lambda b: (b, 0, 0)),
        ],
        out_specs=pl.BlockSpec((1, S, N), lambda b: (b, 0, 0)),
        out_shape=jax.ShapeDtypeStruct((B, S, N), jnp.float32),
    )(xyz.reshape(B, 3, N), nx.reshape(B, 1, S), ny.reshape(B, 1, S),
      nz.reshape(B, 1, S), qc.reshape(B, 1, S))


def _l1_body(x_ref, w_ref, b_ref, y_ref, st_ref, a1, a2):
    i = pl.program_id(0)
    Xb = x_ref[0, 0]
    Y = jax.lax.dot_general(w_ref[...], Xb, (((1,), (0,)), ((), ())),
                            preferred_element_type=jnp.float32)
    Y = Y + b_ref[...]
    y_ref[0, 0] = Y
    s1 = jnp.sum(Y, axis=1, keepdims=True)
    s2 = jnp.sum(Y * Y, axis=1, keepdims=True)

    @pl.when(i == 0)
    def _():
        a1[...] = s1
        a2[...] = s2

    @pl.when(i > 0)
    def _():
        a1[...] += s1
        a2[...] += s2

    @pl.when(i == B * K - 1)
    def _():
        st_ref[:, 0:1] = a1[...] * 1.0
        st_ref[:, 1:2] = a2[...] * 1.0


def _run_l1(x0, W0, b0):
    cout = W0.shape[0]
    return pl.pallas_call(
        _l1_body,
        grid=(B * K,),
        in_specs=[
            pl.BlockSpec((1, 1, CIN, S), lambda i: (i // K, i % K, 0, 0)),
            pl.BlockSpec((cout, CIN), lambda i: (0, 0)),
            pl.BlockSpec((cout, 1), lambda i: (0, 0)),
        ],
        out_specs=[
            pl.BlockSpec((1, 1, cout, S), lambda i: (i // K, i % K, 0, 0)),
            pl.BlockSpec((cout, 2), lambda i: (0, 0)),
        ],
        out_shape=[
            jax.ShapeDtypeStruct((B, K, cout, S), jnp.float32),
            jax.ShapeDtypeStruct((cout, 2), jnp.float32),
        ],
        scratch_shapes=[
            pltpu.VMEM((cout, 1), jnp.float32),
            pltpu.VMEM((cout, 1), jnp.float32),
        ],
    )(x0, W0, b0)


def _norm_prev(Yb, st_ref, g_ref, be_ref):
    m = st_ref[:, 0:1] * (1.0 / NTOT)
    v = st_ref[:, 1:2] * (1.0 / NTOT) - m * m
    xh = g_ref[...] * (Yb - m) / jnp.sqrt(v + EPS) + be_ref[...]
    return jnp.maximum(xh, 0.0)


def _mid_body(y_ref, st_ref, g_ref, be_ref, w_ref, b_ref, y2_ref, st2_ref,
              a1, a2):
    i = pl.program_id(0)
    Xb = _norm_prev(y_ref[0, 0], st_ref, g_ref, be_ref)
    Y = jax.lax.dot_general(w_ref[...], Xb, (((1,), (0,)), ((), ())),
                            preferred_element_type=jnp.float32)
    Y = Y + b_ref[...]
    y2_ref[0, 0] = Y
    s1 = jnp.sum(Y, axis=1, keepdims=True)
    s2 = jnp.sum(Y * Y, axis=1, keepdims=True)

    @pl.when(i == 0)
    def _():
        a1[...] = s1
        a2[...] = s2

    @pl.when(i > 0)
    def _():
        a1[...] += s1
        a2[...] += s2

    @pl.when(i == B * K - 1)
    def _():
        st2_ref[:, 0:1] = a1[...] * 1.0
        st2_ref[:, 1:2] = a2[...] * 1.0


def _run_mid(y1, st1, g_prev, be_prev, W, b):
    cin = y1.shape[2]
    cout = W.shape[0]
    return pl.pallas_call(
        _mid_body,
        grid=(B * K,),
        in_specs=[
            pl.BlockSpec((1, 1, cin, S), lambda i: (i // K, i % K, 0, 0)),
            pl.BlockSpec((cin, 2), lambda i: (0, 0)),
            pl.BlockSpec((cin, 1), lambda i: (0, 0)),
            pl.BlockSpec((cin, 1), lambda i: (0, 0)),
            pl.BlockSpec((cout, cin), lambda i: (0, 0)),
            pl.BlockSpec((cout, 1), lambda i: (0, 0)),
        ],
        out_specs=[
            pl.BlockSpec((1, 1, cout, S), lambda i: (i // K, i % K, 0, 0)),
            pl.BlockSpec((cout, 2), lambda i: (0, 0)),
        ],
        out_shape=[
            jax.ShapeDtypeStruct((B, K, cout, S), jnp.float32),
            jax.ShapeDtypeStruct((cout, 2), jnp.float32),
        ],
        scratch_shapes=[
            pltpu.VMEM((cout, 1), jnp.float32),
            pltpu.VMEM((cout, 1), jnp.float32),
        ],
    )(y1, st1, g_prev, be_prev, W, b)


def _l3_body(y_ref, st_ref, g_ref, be_ref, w_ref, b_ref,
             mx_ref, mn_ref, st3_ref, a1, a2):
    i = pl.program_id(0)
    k = i % K
    Xb = _norm_prev(y_ref[0, 0], st_ref, g_ref, be_ref)
    Y = jax.lax.dot_general(w_ref[...], Xb, (((1,), (0,)), ((), ())),
                            preferred_element_type=jnp.float32)
    Y = Y + b_ref[...]
    s1 = jnp.sum(Y, axis=1, keepdims=True)
    s2 = jnp.sum(Y * Y, axis=1, keepdims=True)

    @pl.when(k == 0)
    def _():
        mx_ref[0] = Y
        mn_ref[0] = Y

    @pl.when(k > 0)
    def _():
        mx_ref[0] = jnp.maximum(mx_ref[0], Y)
        mn_ref[0] = jnp.minimum(mn_ref[0], Y)

    @pl.when(i == 0)
    def _():
        a1[...] = s1
        a2[...] = s2

    @pl.when(i > 0)
    def _():
        a1[...] += s1
        a2[...] += s2

    @pl.when(i == B * K - 1)
    def _():
        st3_ref[:, 0:1] = a1[...] * 1.0
        st3_ref[:, 1:2] = a2[...] * 1.0


def _run_l3(y2, st2, g_prev, be_prev, W, b):
    cin = y2.shape[2]
    cout = W.shape[0]
    return pl.pallas_call(
        _l3_body,
        grid=(B * K,),
        in_specs=[
            pl.BlockSpec((1, 1, cin, S), lambda i: (i // K, i % K, 0, 0)),
            pl.BlockSpec((cin, 2), lambda i: (0, 0)),
            pl.BlockSpec((cin, 1), lambda i: (0, 0)),
            pl.BlockSpec((cin, 1), lambda i: (0, 0)),
            pl.BlockSpec((cout, cin), lambda i: (0, 0)),
            pl.BlockSpec((cout, 1), lambda i: (0, 0)),
        ],
        out_specs=[
            pl.BlockSpec((1, cout, S), lambda i: (i // K, 0, 0)),
            pl.BlockSpec((1, cout, S), lambda i: (i // K, 0, 0)),
            pl.BlockSpec((cout, 2), lambda i: (0, 0)),
        ],
        out_shape=[
            jax.ShapeDtypeStruct((B, cout, S), jnp.float32),
            jax.ShapeDtypeStruct((B, cout, S), jnp.float32),
            jax.ShapeDtypeStruct((cout, 2), jnp.float32),
        ],
        scratch_shapes=[
            pltpu.VMEM((cout, 1), jnp.float32),
            pltpu.VMEM((cout, 1), jnp.float32),
        ],
    )(y2, st2, g_prev, be_prev, W, b)


def _fin_body(mx_ref, mn_ref, st_ref, g_ref, be_ref, o_ref):
    m = st_ref[:, 0:1] * (1.0 / NTOT)
    v = st_ref[:, 1:2] * (1.0 / NTOT) - m * m
    g = g_ref[...]
    val = jnp.where(g >= 0.0, mx_ref[0], mn_ref[0])
    xh = g * (val - m) / jnp.sqrt(v + EPS) + be_ref[...]
    o_ref[0] = jnp.maximum(xh, 0.0)


def _run_fin(mx, mn, st3, g, be):
    cout = mx.shape[1]
    return pl.pallas_call(
        _fin_body,
        grid=(B,),
        in_specs=[
            pl.BlockSpec((1, cout, S), lambda b: (b, 0, 0)),
            pl.BlockSpec((1, cout, S), lambda b: (b, 0, 0)),
            pl.BlockSpec((cout, 2), lambda b: (0, 0)),
            pl.BlockSpec((cout, 1), lambda b: (0, 0)),
            pl.BlockSpec((cout, 1), lambda b: (0, 0)),
        ],
        out_specs=pl.BlockSpec((1, cout, S), lambda b: (b, 0, 0)),
        out_shape=jax.ShapeDtypeStruct((B, cout, S), jnp.float32),
    )(mx, mn, st3, g, be)


def kernel(xyz, points, W0, b0, g0, be0, W1, b1, g1, be1, W2, b2, g2, be2):
    far0 = jax.random.randint(jax.random.key(42), (B,), 0, N).astype(
        jnp.int32).reshape(B, 1)

    cent, nx, ny, nz, qc = _run_fps(xyz, far0)
    new_xyz = jnp.stack([nx, ny, nz], axis=1)

    dist = _run_dist(xyz, nx, ny, nz, qc)
    _, idxk = jax.lax.top_k(-dist, K + 1)
    idx = idxk[:, :, 1:]

    xyz_t = jnp.transpose(xyz, (0, 2, 1))
    pts_t = jnp.transpose(points, (0, 2, 1))
    gx = jax.vmap(lambda p, i: p[i])(xyz_t, idx)
    gf = jax.vmap(lambda p, i: p[i])(pts_t, idx)
    gn = gx - jnp.stack([nx, ny, nz], axis=-1)[:, :, None, :]
    new_fea = jnp.concatenate([gn, gf], axis=-1)
    x0 = jnp.transpose(new_fea, (0, 2, 3, 1))

    y1, st1 = _run_l1(x0, W0, b0.reshape(-1, 1))
    y2, st2 = _run_mid(y1, st1, g0.reshape(-1, 1), be0.reshape(-1, 1),
                       W1, b1.reshape(-1, 1))
    mx, mn, st3 = _run_l3(y2, st2, g1.reshape(-1, 1), be1.reshape(-1, 1),
                          W2, b2.reshape(-1, 1))
    new_points = _run_fin(mx, mn, st3, g2.reshape(-1, 1), be2.reshape(-1, 1))
    return (new_xyz, new_points)


# speedup vs baseline: 12.1247x; 5.9988x over previous
import functools

import jax
import jax.numpy as jnp
import numpy as np
from jax import lax
from jax.experimental import pallas as pl
from jax.experimental.pallas import tpu as pltpu
from jax.experimental.pallas import tpu_sc as plsc

B = 16
N = 2048
D = 64
S = 512
K = 32
CIN = 67
CPAD = 128
NTOT = float(B * K * S)
EPS = 1e-5

NW = 32
ROWS = B * K * S
RPW = ROWS // NW
CHUNK = 128
NCH = RPW // CHUNK


def _fps_body(xyz_ref, far0_ref, cent_ref, nx_ref, ny_ref, nz_ref, qc_ref):
    X = xyz_ref[:, 0, :]
    Y = xyz_ref[:, 1, :]
    Z = xyz_ref[:, 2, :]
    jlane = jax.lax.broadcasted_iota(jnp.int32, (B, N), 1)
    siota = jax.lax.broadcasted_iota(jnp.int32, (B, S), 1)
    far0 = far0_ref[...]

    dist0 = jnp.full((B, N), 1e10, dtype=jnp.float32)
    cent0 = jnp.zeros((B, S), dtype=jnp.int32)
    zf = jnp.zeros((B, S), dtype=jnp.float32)

    def body(i, c):
        dist, far, cent, nx, ny, nz, qc = c
        mask = jlane == far
        cx = jnp.sum(jnp.where(mask, X, 0.0), axis=1, keepdims=True)
        cy = jnp.sum(jnp.where(mask, Y, 0.0), axis=1, keepdims=True)
        cz = jnp.sum(jnp.where(mask, Z, 0.0), axis=1, keepdims=True)
        sel = siota == i
        cent = jnp.where(sel, far, cent)
        nx = jnp.where(sel, cx, nx)
        ny = jnp.where(sel, cy, ny)
        nz = jnp.where(sel, cz, nz)
        qc = jnp.where(sel, (cx * cx + cy * cy) + cz * cz, qc)
        dx = X - cx
        dy = Y - cy
        dz = Z - cz
        d = dx * dx + dy * dy + dz * dz
        dist = jnp.where(d < dist, d, dist)
        m = jnp.max(dist, axis=1, keepdims=True)
        far = jnp.min(jnp.where(dist == m, jlane, N), axis=1, keepdims=True)
        return (dist, far, cent, nx, ny, nz, qc)

    _, _, cent, nx, ny, nz, qc = jax.lax.fori_loop(
        0, S, body, (dist0, far0, cent0, zf, zf, zf, zf))
    cent_ref[...] = cent
    nx_ref[...] = nx
    ny_ref[...] = ny
    nz_ref[...] = nz
    qc_ref[...] = qc


def _run_fps(xyz, far0):
    outs = (
        jax.ShapeDtypeStruct((B, S), jnp.int32),
        jax.ShapeDtypeStruct((B, S), jnp.float32),
        jax.ShapeDtypeStruct((B, S), jnp.float32),
        jax.ShapeDtypeStruct((B, S), jnp.float32),
        jax.ShapeDtypeStruct((B, S), jnp.float32),
    )
    return pl.pallas_call(_fps_body, out_shape=outs)(xyz, far0)


def _knn_body(xyz_ref, nx_ref, ny_ref, nz_ref, qc_ref, idx_ref, d_scr):
    X = xyz_ref[0, 0, :][None, :]
    Y = xyz_ref[0, 1, :][None, :]
    Z = xyz_ref[0, 2, :][None, :]
    qj = (X * X + Y * Y) + Z * Z
    cx = nx_ref[0, 0, :][:, None]
    cy = ny_ref[0, 0, :][:, None]
    cz = nz_ref[0, 0, :][:, None]
    qc = qc_ref[0, 0, :][:, None]
    bf = jnp.bfloat16
    f32 = jnp.float32
    Xb = X.astype(bf).astype(f32)
    Yb = Y.astype(bf).astype(f32)
    Zb = Z.astype(bf).astype(f32)
    cxb = cx.astype(bf).astype(f32)
    cyb = cy.astype(bf).astype(f32)
    czb = cz.astype(bf).astype(f32)
    inner = (cxb * Xb + cyb * Yb) + czb * Zb
    d_scr[...] = (-2.0 * inner + qj) + qc

    jlane = jax.lax.broadcasted_iota(jnp.int32, (S, N), 1)
    tio = jax.lax.broadcasted_iota(jnp.int32, (S, 64), 1)

    def body(t, acc):
        Dc = d_scr[...]
        m = jnp.min(Dc, axis=1, keepdims=True)
        j = jnp.min(jnp.where(Dc == m, jlane, N), axis=1, keepdims=True)
        d_scr[...] = jnp.where(jlane == j, 1e30, Dc)
        return jnp.where(tio == t, j, acc)

    idx_ref[0] = jax.lax.fori_loop(
        0, K + 1, body, jnp.zeros((S, 64), jnp.int32))


def _run_knn(xyz, nx, ny, nz, qc):
    return pl.pallas_call(
        _knn_body,
        grid=(B,),
        in_specs=[
            pl.BlockSpec((1, 3, N), lambda b: (b, 0, 0)),
            pl.BlockSpec((1, 1, S), lambda b: (b, 0, 0)),
            pl.BlockSpec((1, 1, S), lambda b: (b, 0, 0)),
            pl.BlockSpec((1, 1, S), lambda b: (b, 0, 0)),
            pl.BlockSpec((1, 1, S), lambda b: (b, 0, 0)),
        ],
        out_specs=pl.BlockSpec((1, S, 64), lambda b: (b, 0, 0)),
        out_shape=jax.ShapeDtypeStruct((B, S, 64), jnp.int32),
        scratch_shapes=[pltpu.VMEM((S, N), jnp.float32)],
    )(xyz, nx.reshape(B, 1, S), ny.reshape(B, 1, S),
      nz.reshape(B, 1, S), qc.reshape(B, 1, S))


def _sc_gather_body(src_ref, idxf_ref, out_ref, idx_v, bufa, bufb, sema, semb):
    wid = lax.axis_index("s") * 2 + lax.axis_index("c")
    base = wid * RPW
    pltpu.sync_copy(idxf_ref.at[pl.ds(base, RPW)], idx_v)
    pltpu.make_async_copy(
        src_ref.at[idx_v.at[pl.ds(0, CHUNK)]], bufa, sema).start()

    def body(i, carry):
        off = 2 * i * CHUNK
        pltpu.make_async_copy(
            src_ref.at[idx_v.at[pl.ds(off, CHUNK)]], bufa, sema).wait()
        hb = pltpu.make_async_copy(
            src_ref.at[idx_v.at[pl.ds(off + CHUNK, CHUNK)]], bufb, semb)
        hb.start()
        pltpu.sync_copy(bufa, out_ref.at[pl.ds(base + off, CHUNK)])
        hb.wait()

        @pl.when(i < NCH // 2 - 1)
        def _():
            pltpu.make_async_copy(
                src_ref.at[idx_v.at[pl.ds(off + 2 * CHUNK, CHUNK)]],
                bufa, sema).start()

        pltpu.sync_copy(bufb, out_ref.at[pl.ds(base + off + CHUNK, CHUNK)])
        return carry

    lax.fori_loop(0, NCH // 2, body, 0)


_sc_gather = functools.partial(
    pl.kernel,
    out_type=jax.ShapeDtypeStruct((ROWS, CPAD), jnp.float32),
    mesh=plsc.VectorSubcoreMesh(core_axis_name="c", subcore_axis_name="s"),
    scratch_types=[
        pltpu.VMEM((RPW,), jnp.int32),
        pltpu.VMEM((CHUNK, CPAD), jnp.float32),
        pltpu.VMEM((CHUNK, CPAD), jnp.float32),
        pltpu.SemaphoreType.DMA,
        pltpu.SemaphoreType.DMA,
    ],
)(_sc_gather_body)


def _l1_body(x_ref, c_ref, w_ref, b_ref, y_ref, st_ref, a1, a2):
    i = pl.program_id(0)
    Xb = x_ref[0, 0]
    C = c_ref[0]
    Cp = jnp.concatenate(
        [C, jnp.zeros((S, CPAD - 3), jnp.float32)], axis=1)
    Xs = Xb - Cp
    Y = jax.lax.dot_general(w_ref[...], Xs, (((1,), (1,)), ((), ())),
                            preferred_element_type=jnp.float32)
    Y = Y + b_ref[...]
    y_ref[0, 0] = Y
    s1 = jnp.sum(Y, axis=1, keepdims=True)
    s2 = jnp.sum(Y * Y, axis=1, keepdims=True)

    @pl.when(i == 0)
    def _():
        a1[...] = s1
        a2[...] = s2

    @pl.when(i > 0)
    def _():
        a1[...] += s1
        a2[...] += s2

    @pl.when(i == B * K - 1)
    def _():
        st_ref[:, 0:1] = a1[...] * 1.0
        st_ref[:, 1:2] = a2[...] * 1.0


def _run_l1(g, cmat, W0p, b0):
    cout = W0p.shape[0]
    return pl.pallas_call(
        _l1_body,
        grid=(B * K,),
        in_specs=[
            pl.BlockSpec((1, 1, S, CPAD), lambda i: (i // K, i % K, 0, 0)),
            pl.BlockSpec((1, S, 3), lambda i: (i // K, 0, 0)),
            pl.BlockSpec((cout, CPAD), lambda i: (0, 0)),
            pl.BlockSpec((cout, 1), lambda i: (0, 0)),
        ],
        out_specs=[
            pl.BlockSpec((1, 1, cout, S), lambda i: (i // K, i % K, 0, 0)),
            pl.BlockSpec((cout, 2), lambda i: (0, 0)),
        ],
        out_shape=[
            jax.ShapeDtypeStruct((B, K, cout, S), jnp.float32),
            jax.ShapeDtypeStruct((cout, 2), jnp.float32),
        ],
        scratch_shapes=[
            pltpu.VMEM((cout, 1), jnp.float32),
            pltpu.VMEM((cout, 1), jnp.float32),
        ],
    )(g, cmat, W0p, b0)


def _norm_prev(Yb, st_ref, g_ref, be_ref):
    m = st_ref[:, 0:1] * (1.0 / NTOT)
    v = st_ref[:, 1:2] * (1.0 / NTOT) - m * m
    xh = g_ref[...] * (Yb - m) / jnp.sqrt(v + EPS) + be_ref[...]
    return jnp.maximum(xh, 0.0)


def _mid_body(y_ref, st_ref, g_ref, be_ref, w_ref, b_ref, y2_ref, st2_ref,
              a1, a2):
    i = pl.program_id(0)
    Xb = _norm_prev(y_ref[0, 0], st_ref, g_ref, be_ref)
    Y = jax.lax.dot_general(w_ref[...], Xb, (((1,), (0,)), ((), ())),
                            preferred_element_type=jnp.float32)
    Y = Y + b_ref[...]
    y2_ref[0, 0] = Y
    s1 = jnp.sum(Y, axis=1, keepdims=True)
    s2 = jnp.sum(Y * Y, axis=1, keepdims=True)

    @pl.when(i == 0)
    def _():
        a1[...] = s1
        a2[...] = s2

    @pl.when(i > 0)
    def _():
        a1[...] += s1
        a2[...] += s2

    @pl.when(i == B * K - 1)
    def _():
        st2_ref[:, 0:1] = a1[...] * 1.0
        st2_ref[:, 1:2] = a2[...] * 1.0


def _run_mid(y1, st1, g_prev, be_prev, W, b):
    cin = y1.shape[2]
    cout = W.shape[0]
    return pl.pallas_call(
        _mid_body,
        grid=(B * K,),
        in_specs=[
            pl.BlockSpec((1, 1, cin, S), lambda i: (i // K, i % K, 0, 0)),
            pl.BlockSpec((cin, 2), lambda i: (0, 0)),
            pl.BlockSpec((cin, 1), lambda i: (0, 0)),
            pl.BlockSpec((cin, 1), lambda i: (0, 0)),
            pl.BlockSpec((cout, cin), lambda i: (0, 0)),
            pl.BlockSpec((cout, 1), lambda i: (0, 0)),
        ],
        out_specs=[
            pl.BlockSpec((1, 1, cout, S), lambda i: (i // K, i % K, 0, 0)),
            pl.BlockSpec((cout, 2), lambda i: (0, 0)),
        ],
        out_shape=[
            jax.ShapeDtypeStruct((B, K, cout, S), jnp.float32),
            jax.ShapeDtypeStruct((cout, 2), jnp.float32),
        ],
        scratch_shapes=[
            pltpu.VMEM((cout, 1), jnp.float32),
            pltpu.VMEM((cout, 1), jnp.float32),
        ],
    )(y1, st1, g_prev, be_prev, W, b)


def _l3_body(y_ref, st_ref, g_ref, be_ref, w_ref, b_ref,
             mx_ref, mn_ref, st3_ref, a1, a2):
    i = pl.program_id(0)
    k = i % K
    Xb = _norm_prev(y_ref[0, 0], st_ref, g_ref, be_ref)
    Y = jax.lax.dot_general(w_ref[...], Xb, (((1,), (0,)), ((), ())),
                            preferred_element_type=jnp.float32)
    Y = Y + b_ref[...]
    s1 = jnp.sum(Y, axis=1, keepdims=True)
    s2 = jnp.sum(Y * Y, axis=1, keepdims=True)

    @pl.when(k == 0)
    def _():
        mx_ref[0] = Y
        mn_ref[0] = Y

    @pl.when(k > 0)
    def _():
        mx_ref[0] = jnp.maximum(mx_ref[0], Y)
        mn_ref[0] = jnp.minimum(mn_ref[0], Y)

    @pl.when(i == 0)
    def _():
        a1[...] = s1
        a2[...] = s2

    @pl.when(i > 0)
    def _():
        a1[...] += s1
        a2[...] += s2

    @pl.when(i == B * K - 1)
    def _():
        st3_ref[:, 0:1] = a1[...] * 1.0
        st3_ref[:, 1:2] = a2[...] * 1.0


def _run_l3(y2, st2, g_prev, be_prev, W, b):
    cin = y2.shape[2]
    cout = W.shape[0]
    return pl.pallas_call(
        _l3_body,
        grid=(B * K,),
        in_specs=[
            pl.BlockSpec((1, 1, cin, S), lambda i: (i // K, i % K, 0, 0)),
            pl.BlockSpec((cin, 2), lambda i: (0, 0)),
            pl.BlockSpec((cin, 1), lambda i: (0, 0)),
            pl.BlockSpec((cin, 1), lambda i: (0, 0)),
            pl.BlockSpec((cout, cin), lambda i: (0, 0)),
            pl.BlockSpec((cout, 1), lambda i: (0, 0)),
        ],
        out_specs=[
            pl.BlockSpec((1, cout, S), lambda i: (i // K, 0, 0)),
            pl.BlockSpec((1, cout, S), lambda i: (i // K, 0, 0)),
            pl.BlockSpec((cout, 2), lambda i: (0, 0)),
        ],
        out_shape=[
            jax.ShapeDtypeStruct((B, cout, S), jnp.float32),
            jax.ShapeDtypeStruct((B, cout, S), jnp.float32),
            jax.ShapeDtypeStruct((cout, 2), jnp.float32),
        ],
        scratch_shapes=[
            pltpu.VMEM((cout, 1), jnp.float32),
            pltpu.VMEM((cout, 1), jnp.float32),
        ],
    )(y2, st2, g_prev, be_prev, W, b)


def _fin_body(mx_ref, mn_ref, st_ref, g_ref, be_ref, o_ref):
    m = st_ref[:, 0:1] * (1.0 / NTOT)
    v = st_ref[:, 1:2] * (1.0 / NTOT) - m * m
    g = g_ref[...]
    val = jnp.where(g >= 0.0, mx_ref[0], mn_ref[0])
    xh = g * (val - m) / jnp.sqrt(v + EPS) + be_ref[...]
    o_ref[0] = jnp.maximum(xh, 0.0)


def _run_fin(mx, mn, st3, g, be):
    cout = mx.shape[1]
    return pl.pallas_call(
        _fin_body,
        grid=(B,),
        in_specs=[
            pl.BlockSpec((1, cout, S), lambda b: (b, 0, 0)),
            pl.BlockSpec((1, cout, S), lambda b: (b, 0, 0)),
            pl.BlockSpec((cout, 2), lambda b: (0, 0)),
            pl.BlockSpec((cout, 1), lambda b: (0, 0)),
            pl.BlockSpec((cout, 1), lambda b: (0, 0)),
        ],
        out_specs=pl.BlockSpec((1, cout, S), lambda b: (b, 0, 0)),
        out_shape=jax.ShapeDtypeStruct((B, cout, S), jnp.float32),
    )(mx, mn, st3, g, be)


def kernel(xyz, points, W0, b0, g0, be0, W1, b1, g1, be1, W2, b2, g2, be2):
    far0 = jax.random.randint(jax.random.key(42), (B,), 0, N).astype(
        jnp.int32).reshape(B, 1)

    cent, nx, ny, nz, qc = _run_fps(xyz, far0)
    new_xyz = jnp.stack([nx, ny, nz], axis=1)

    idx64 = _run_knn(xyz, nx, ny, nz, qc)
    idx = idx64[:, :, 1:K + 1]

    xyz_t = jnp.transpose(xyz, (0, 2, 1))
    pts_t = jnp.transpose(points, (0, 2, 1))
    src = jnp.concatenate(
        [xyz_t, pts_t, jnp.zeros((B, N, CPAD - CIN), jnp.float32)],
        axis=-1).reshape(B * N, CPAD)
    idxf = (jnp.transpose(idx, (0, 2, 1)) +
            (jnp.arange(B, dtype=jnp.int32) * N)[:, None, None]
            ).reshape(ROWS).astype(jnp.int32)
    g = _sc_gather(src, idxf).reshape(B, K, S, CPAD)

    W0p = jnp.concatenate(
        [W0, jnp.zeros((W0.shape[0], CPAD - CIN), jnp.float32)], axis=1)
    cmat = jnp.stack([nx, ny, nz], axis=-1)
    y1, st1 = _run_l1(g, cmat, W0p, b0.reshape(-1, 1))
    y2, st2 = _run_mid(y1, st1, g0.reshape(-1, 1), be0.reshape(-1, 1),
                       W1, b1.reshape(-1, 1))
    mx, mn, st3 = _run_l3(y2, st2, g1.reshape(-1, 1), be1.reshape(-1, 1),
                          W2, b2.reshape(-1, 1))
    new_points = _run_fin(mx, mn, st3, g2.reshape(-1, 1), be2.reshape(-1, 1))
    return (new_xyz, new_points)
